# baseline (speedup 1.0000x reference)
import jax
import jax.numpy as jnp
from jax.experimental import pallas as pl

_BATCH = 16384
_DIMS = [(24, 1052), (1052, 526), (526, 256), (256, 128), (128, 64), (64, 4)]
_NLAYERS = len(_DIMS)
_EPS = 1e-5
_NCODES = 64

_WLANES = 1052


def _ru(n, m):
    return (n + m - 1) // m * m


_WOFF = []
_r = 0
for _dout in [d for _, d in _DIMS]:
    _WOFF.append(_r)
    _r = _ru(_r + _dout, 16)
_EOFF = []
for _t in range(3):
    _EOFF.append(_r)
    _r += 16
_WROWS = _r

_BOFF = []
_c = 0
for _dout in [d for _, d in _DIMS]:
    _BOFF.append(_c)
    _c += _ru(_dout, 128)
_BLANES = _c


def _body(xt_ref, wall_ref, ball_ref, out_ref):
    xt = xt_ref[...]
    code = xt[0:1, :] * 16 + xt[1:2, :] * 4 + xt[2:3, :]
    sub = jax.lax.broadcasted_iota(jnp.int32, (_NCODES, _BATCH), 0)
    oht = (code == sub).astype(jnp.bfloat16)

    ones = jnp.ones((_BATCH, 1), jnp.bfloat16)
    counts = jnp.dot(oht, ones, preferred_element_type=jnp.float32)
    w = counts * (1.0 / _BATCH)

    row = jax.lax.broadcasted_iota(jnp.int32, (_NCODES, 4), 0)
    col = jax.lax.broadcasted_iota(jnp.int32, (_NCODES, 4), 1)
    parts = []
    for t, shift in enumerate((4, 2, 0)):
        sel = (jnp.right_shift(row, shift) & 3) == col
        et = wall_ref[_EOFF[t]:_EOFF[t] + 4, 0:8]
        parts.append(jnp.dot(sel.astype(jnp.bfloat16), et,
                             preferred_element_type=jnp.float32))
    h = jnp.concatenate(parts, axis=1)

    for i in range(_NLAYERS):
        din, dout = _DIMS[i]
        wi = wall_ref[_WOFF[i]:_WOFF[i] + dout, 0:din]
        bi = ball_ref[0:1, _BOFF[i]:_BOFF[i] + dout]
        z = jax.lax.dot_general(
            h.astype(jnp.bfloat16), wi,
            dimension_numbers=(((1,), (1,)), ((), ())),
            preferred_element_type=jnp.float32) + bi
        if i < _NLAYERS - 1:
            gi = ball_ref[1:2, _BOFF[i]:_BOFF[i] + dout]
            bei = ball_ref[2:3, _BOFF[i]:_BOFF[i] + dout]
            r = jnp.maximum(z, 0.0)
            m = jnp.sum(w * r, axis=0, keepdims=True)
            d = r - m
            v = jnp.sum(w * (d * d), axis=0, keepdims=True)
            h = d * (gi * jax.lax.rsqrt(v + _EPS)) + bei
        else:
            h = z

    h_hi = h.astype(jnp.bfloat16).astype(jnp.float32)
    h_lo = h - h_hi
    hl = jnp.concatenate([h_hi, h_lo], axis=1)
    hlt = jnp.transpose(hl).astype(jnp.bfloat16)
    g8 = jnp.dot(hlt, oht, preferred_element_type=jnp.float32)
    out_ref[...] = g8[0:4, :] + g8[4:8, :]


def kernel(params, x):
    wparts = []
    for i, (din, dout) in enumerate(_DIMS):
        wi = params[f"W{i}"].astype(jnp.bfloat16)
        wparts.append(jnp.pad(wi, ((0, _ru(dout, 16) - dout),
                                   (0, _WLANES - din))))
    for t in range(3):
        et = params[f"E{t}"].astype(jnp.bfloat16)
        wparts.append(jnp.pad(et, ((0, 12), (0, _WLANES - 8))))
    wall = jnp.concatenate(wparts, axis=0)

    brows = []
    for name, n in (("b", _NLAYERS), ("g", _NLAYERS - 1), ("be", _NLAYERS - 1)):
        pieces = []
        for i in range(_NLAYERS):
            dout = _DIMS[i][1]
            vec = (params[f"{name}{i}"] if i < n
                   else jnp.zeros((dout,), jnp.float32))
            pieces.append(jnp.pad(vec, (0, _ru(dout, 128) - dout)))
        brows.append(jnp.concatenate(pieces))
    ball = jnp.stack(brows)

    out_t = pl.pallas_call(
        _body,
        out_shape=jax.ShapeDtypeStruct((4, _BATCH), jnp.float32),
    )(x.T, wall, ball)
    return out_t.T
